# baseline (speedup 1.0000x reference)
import jax
import jax.numpy as jnp
from jax.experimental import pallas as pl
from jax.experimental.pallas import tpu as pltpu

_NUM_CHUNKS = 8


def _dma_copy(x_hbm, o_hbm, sem):
    n = x_hbm.shape[0]
    rows = n // _NUM_CHUNKS
    copies = [
        pltpu.make_async_copy(
            x_hbm.at[pl.ds(i * rows, rows)],
            o_hbm.at[pl.ds(i * rows, rows)],
            sem.at[i],
        )
        for i in range(_NUM_CHUNKS)
    ]
    for c in copies:
        c.start()
    for c in copies:
        c.wait()


def kernel(embedding_weight):
    n, d = embedding_weight.shape
    return pl.pallas_call(
        _dma_copy,
        out_shape=jax.ShapeDtypeStruct((n, d), embedding_weight.dtype),
        in_specs=[pl.BlockSpec(memory_space=pl.ANY)],
        out_specs=pl.BlockSpec(memory_space=pl.ANY),
        scratch_shapes=[pltpu.SemaphoreType.DMA((_NUM_CHUNKS,))],
    )(embedding_weight)


# speedup vs baseline: 30.2915x; 30.2915x over previous
import jax
import jax.numpy as jnp
from jax.experimental import pallas as pl
from jax.experimental.pallas import tpu as pltpu


def _copy_block(x_ref, o_ref):
    o_ref[...] = x_ref[...]


def kernel(embedding_weight):
    n, d = embedding_weight.shape
    block_rows = 2000
    return pl.pallas_call(
        _copy_block,
        out_shape=jax.ShapeDtypeStruct((n, d), embedding_weight.dtype),
        grid=(n // block_rows,),
        in_specs=[pl.BlockSpec((block_rows, d), lambda i: (i, 0))],
        out_specs=pl.BlockSpec((block_rows, d), lambda i: (i, 0)),
        compiler_params=pltpu.CompilerParams(
            dimension_semantics=("parallel",),
        ),
    )(embedding_weight)
